# baseline (speedup 1.0000x reference)
import jax
import jax.numpy as jnp
from jax import lax
from jax.experimental import pallas as pl
from jax.experimental.pallas import tpu as pltpu
from jax.experimental.pallas import tpu_sc as plsc

N = 10000
E = 320000
D = 128
H = 2
C = 64
NUM_CLASSES = 16

K = 128
E_PAD = 327680
EPT = E_PAD // 32
NCHUNK = EPT // K
RPT = N // 16
CH0 = 58
CH1 = 102
NEG_SLOPE = 0.2

_SC_PARAMS = pltpu.CompilerParams(
    use_tc_tiling_on_sc=False, needs_layout_passes=False)


def _leaky(x):
    return jnp.where(x >= 0, x, NEG_SLOPE * x)


def _pre_body(x_ref, w_ref, asv_ref, adv_ref, h_ref, atab_ref):
    hb = jnp.dot(x_ref[...], w_ref[0], preferred_element_type=jnp.float32)
    h_ref[...] = hb
    asv = asv_ref[0]
    adv = adv_ref[0]
    cols = []
    for g in range(2):
        sl = slice(g * 64, (g + 1) * 64)
        cols.append(jnp.sum(hb[:, sl] * asv[:, sl], axis=1))
    for g in range(2):
        sl = slice(g * 64, (g + 1) * 64)
        cols.append(jnp.sum(hb[:, sl] * adv[:, sl], axis=1))
    z = jnp.zeros_like(cols[0])
    atab_ref[...] = jnp.stack(cols + [z, z, z, z], axis=1)


def _pre_call(x, W3, a_src3, a_dst3):
    nb = 10
    bs = N // nb
    return pl.pallas_call(
        _pre_body,
        grid=(3 * nb,),
        in_specs=[
            pl.BlockSpec((bs, D), lambda i: (i % nb, 0)),
            pl.BlockSpec((1, D, D), lambda i: (i // nb, 0, 0)),
            pl.BlockSpec((1, 1, D), lambda i: (i // nb, 0, 0)),
            pl.BlockSpec((1, 1, D), lambda i: (i // nb, 0, 0)),
        ],
        out_specs=[
            pl.BlockSpec((bs, D), lambda i: (i, 0)),
            pl.BlockSpec((bs, 8), lambda i: (i, 0)),
        ],
        out_shape=[
            jax.ShapeDtypeStruct((3 * N, D), jnp.float32),
            jax.ShapeDtypeStruct((3 * N, 8), jnp.float32),
        ],
    )(x, W3, a_src3, a_dst3)


def _edges3_body(hb3, atab3, epack, zn, zd,
                 num_out, den_out,
                 ebuf0, ebuf1, sidx0, sidx1, didx0, didx1, didxa0, didxa1,
                 abs0, abs1, abd0, abd1, rows0, rows1, wbuf0, wbuf1,
                 num_acc, den_acc, sem0, sem1):
    cid = lax.axis_index("c")
    sid = lax.axis_index("s")
    iota = lax.iota(jnp.int32, 16)
    cid32 = cid.astype(jnp.int32)
    base_chunk = sid * (CH0 + CH1) + cid32 * CH0
    npairs = jnp.where(cid32 == 0, CH0 // 2, CH1 // 2)
    full = lambda v: jnp.full((16,), v, jnp.int32)
    ebufs = [ebuf0, ebuf1]
    sidxs = [sidx0, sidx1]
    didxs = [didx0, didx1]
    didxas = [didxa0, didxa1]
    abss = [abs0, abs1]
    abds = [abd0, abd1]
    rowss = [rows0, rows1]
    wbufs = [wbuf0, wbuf1]
    sems = [sem0, sem1]

    for br in range(3):
        def zbody(i, carry):
            r0 = sid * RPT + i * 125
            pltpu.sync_copy(zn, num_acc.at[pl.ds(r0, 125)])
            pltpu.sync_copy(zd, den_acc.at[pl.ds(r0, 125)])
            return carry
        lax.fori_loop(0, RPT // 125, zbody, 0)
        plsc.subcore_barrier()

        def fetch(c, p):
            base = (base_chunk + c) * K
            pltpu.sync_copy(epack.at[pl.ds(base, K)], ebufs[p])
            for j in range(K // 16):
                e16 = j * 16 + iota
                s16 = plsc.load_gather(ebufs[p], [e16, full(0)])
                d16 = plsc.load_gather(ebufs[p], [e16, full(1)])
                plsc.store_scatter(sidxs[p], [e16], s16 + br * N)
                plsc.store_scatter(didxs[p], [e16], d16)
                plsc.store_scatter(didxas[p], [e16], d16 + br * N)
            pltpu.async_copy(atab3.at[sidxs[p]], abss[p], sems[p])
            pltpu.async_copy(atab3.at[didxas[p]], abds[p], sems[p])
            pltpu.async_copy(hb3.at[sidxs[p]], rowss[p], sems[p])

        def wait_fetch(p):
            pltpu.make_async_copy(atab3.at[sidxs[p]], abss[p], sems[p]).wait()
            pltpu.make_async_copy(atab3.at[didxas[p]], abds[p], sems[p]).wait()
            pltpu.make_async_copy(hb3.at[sidxs[p]], rowss[p], sems[p]).wait()

        def process(p):
            for j in range(K // 16):
                e16 = j * 16 + iota
                m16 = plsc.load_gather(ebufs[p], [e16, full(2)])
                if br == 0:
                    emask = m16 == 0
                elif br == 1:
                    emask = m16 == 1
                else:
                    emask = m16 <= 1
                for g in range(2):
                    asv = plsc.load_gather(abss[p], [e16, full(g)])
                    adv = plsc.load_gather(abds[p], [e16, full(2 + g)])
                    wv = jnp.exp(_leaky(asv + adv))
                    wv = jnp.where(emask, wv, 0.0)
                    plsc.store_scatter(wbufs[p], [e16, full(g)], wv)

            def ebody(eh, carry2):
                for u in range(4):
                    e = eh * 4 + u
                    fe = jnp.full((16,), e, jnp.int32)
                    for g in range(2):
                        wspl = plsc.load_gather(wbufs[p], [fe, full(g)])
                        for q in range(4):
                            sl = pl.ds(g * 64 + q * 16, 16)
                            rowss[p][e, sl] = rowss[p][e, sl] * wspl
                return carry2
            lax.fori_loop(0, K // 4, ebody, 0)

            pltpu.sync_copy(rowss[p], num_acc.at[didxs[p]], add=True)
            pltpu.sync_copy(wbufs[p], den_acc.at[didxs[p]], add=True)

        def chunk_body(it, carry):
            fetch(it * 2, 0)
            fetch(it * 2 + 1, 1)
            wait_fetch(0)
            process(0)
            wait_fetch(1)
            process(1)
            return carry
        lax.fori_loop(0, npairs, chunk_body, 0)
        plsc.subcore_barrier()

        r0 = sid * RPT
        pltpu.sync_copy(num_acc.at[pl.ds(r0, RPT)],
                        num_out.at[br, cid, pl.ds(r0, RPT)])
        pltpu.sync_copy(den_acc.at[pl.ds(r0, RPT)],
                        den_out.at[br, cid, pl.ds(r0, RPT)])
        plsc.subcore_barrier()


def _edges3_call(hb3, atab3, epack, zn, zd):
    mesh = plsc.VectorSubcoreMesh(core_axis_name="c", subcore_axis_name="s")
    f = pl.kernel(
        _edges3_body,
        mesh=mesh,
        out_type=[
            jax.ShapeDtypeStruct((3, 2, N, D), jnp.float32),
            jax.ShapeDtypeStruct((3, 2, N, 4), jnp.float32),
        ],
        scratch_types=(
            [pltpu.VMEM((K, 4), jnp.int32)] * 2
            + [pltpu.VMEM((K,), jnp.int32)] * 6
            + [pltpu.VMEM((K, 8), jnp.float32)] * 4
            + [pltpu.VMEM((K, D), jnp.float32)] * 2
            + [pltpu.VMEM((K, 4), jnp.float32)] * 2
            + [
                pltpu.VMEM_SHARED((N, D), jnp.float32),
                pltpu.VMEM_SHARED((N, 4), jnp.float32),
                pltpu.SemaphoreType.DMA,
                pltpu.SemaphoreType.DMA,
            ]
        ),
        compiler_params=_SC_PARAMS,
    )
    return f(hb3, atab3, epack, zn, zd)


def _mid_body(n0_ref, n1_ref, n2_ref, n3_ref, n4_ref, n5_ref,
              d0_ref, d1_ref, d2_ref, d3_ref, d4_ref, d5_ref,
              h0_ref, h1_ref, h2_ref, at0_ref, at1_ref, at2_ref,
              bcat_ref, wl_ref, asl_ref, adl_ref,
              h4_ref, atab4_ref):
    nrefs = [n0_ref, n1_ref, n2_ref, n3_ref, n4_ref, n5_ref]
    drefs = [d0_ref, d1_ref, d2_ref, d3_ref, d4_ref, d5_ref]
    hrefs = [h0_ref, h1_ref, h2_ref]
    atrefs = [at0_ref, at1_ref, at2_ref]
    branches = []
    for br in range(3):
        at = atrefs[br][...]
        wsl = jnp.exp(_leaky(at[:, 0:2] + at[:, 2:4]))
        bs = wsl.shape[0]
        wslx = jnp.broadcast_to(wsl[:, :, None], (bs, 2, 64)).reshape(bs, D)
        numf = nrefs[2 * br][0] + nrefs[2 * br + 1][0] + hrefs[br][...] * wslx
        denf = drefs[2 * br][0][:, 0:2] + drefs[2 * br + 1][0][:, 0:2] + wsl
        denx = jnp.broadcast_to(denf[:, :, None], (bs, 2, 64)).reshape(bs, D)
        outb = numf / (denx + 1e-16) \
            + bcat_ref[0, br * D:(br + 1) * D][None, :]
        branches.append(jnp.where(outb > 0, outb, jnp.exp(outb) - 1.0))
    xc = jnp.concatenate(branches, axis=1)
    h4 = jnp.dot(xc, wl_ref[...], preferred_element_type=jnp.float32)
    h4_ref[...] = h4
    asrc4 = jnp.sum(h4 * asl_ref[...], axis=1)
    adst4 = jnp.sum(h4 * adl_ref[...], axis=1)
    z = jnp.zeros_like(asrc4)
    atab4_ref[...] = jnp.stack([asrc4, adst4, z, z, z, z, z, z], axis=1)


def _mid_call(num6, den6, hb3, atab3, bcat, W_last, a_src_last, a_dst_last):
    nb = 10
    bs = N // nb
    nspec = [pl.BlockSpec((1, bs, D), (lambda k: lambda i: (k, i, 0))(k))
             for k in range(6)]
    dspec = [pl.BlockSpec((1, bs, 4), (lambda k: lambda i: (k, i, 0))(k))
             for k in range(6)]
    hspec = [pl.BlockSpec((bs, D), (lambda b: lambda i: (i + b * nb, 0))(b))
             for b in range(3)]
    aspec = [pl.BlockSpec((bs, 8), (lambda b: lambda i: (i + b * nb, 0))(b))
             for b in range(3)]
    return pl.pallas_call(
        _mid_body,
        grid=(nb,),
        in_specs=nspec + dspec + hspec + aspec + [
            pl.BlockSpec((1, 3 * D), lambda i: (0, 0)),
            pl.BlockSpec((3 * D, 16), lambda i: (0, 0)),
            pl.BlockSpec((1, 16), lambda i: (0, 0)),
            pl.BlockSpec((1, 16), lambda i: (0, 0)),
        ],
        out_specs=[
            pl.BlockSpec((bs, 16), lambda i: (i, 0)),
            pl.BlockSpec((bs, 8), lambda i: (i, 0)),
        ],
        out_shape=[
            jax.ShapeDtypeStruct((N, 16), jnp.float32),
            jax.ShapeDtypeStruct((N, 8), jnp.float32),
        ],
    )(*([num6] * 6), *([den6] * 6), *([hb3] * 3), *([atab3] * 3),
      bcat, W_last, a_src_last, a_dst_last)


def _edges4_body(h4, atab4, epack, zn4, zd,
                 num_out, den_out,
                 ebuf0, ebuf1, sidx0, sidx1, didx0, didx1,
                 abs0, abs1, abd0, abd1, rows0, rows1, wbuf0, wbuf1,
                 num_acc, den_acc, sem0, sem1):
    cid = lax.axis_index("c")
    sid = lax.axis_index("s")
    iota = lax.iota(jnp.int32, 16)
    cid32 = cid.astype(jnp.int32)
    base_chunk = sid * (CH0 + CH1) + cid32 * CH0
    npairs = jnp.where(cid32 == 0, CH0 // 2, CH1 // 2)
    full = lambda v: jnp.full((16,), v, jnp.int32)
    ebufs = [ebuf0, ebuf1]
    sidxs = [sidx0, sidx1]
    didxs = [didx0, didx1]
    abss = [abs0, abs1]
    abds = [abd0, abd1]
    rowss = [rows0, rows1]
    wbufs = [wbuf0, wbuf1]
    sems = [sem0, sem1]

    def zbody(i, carry):
        r0 = sid * RPT + i * 125
        pltpu.sync_copy(zn4, num_acc.at[pl.ds(r0, 125)])
        pltpu.sync_copy(zd, den_acc.at[pl.ds(r0, 125)])
        return carry
    lax.fori_loop(0, RPT // 125, zbody, 0)
    plsc.subcore_barrier()

    def fetch(c, p):
        base = (base_chunk + c) * K
        pltpu.sync_copy(epack.at[pl.ds(base, K)], ebufs[p])
        for j in range(K // 16):
            e16 = j * 16 + iota
            s16 = plsc.load_gather(ebufs[p], [e16, full(0)])
            d16 = plsc.load_gather(ebufs[p], [e16, full(1)])
            plsc.store_scatter(sidxs[p], [e16], s16)
            plsc.store_scatter(didxs[p], [e16], d16)
        pltpu.async_copy(atab4.at[sidxs[p]], abss[p], sems[p])
        pltpu.async_copy(atab4.at[didxs[p]], abds[p], sems[p])
        pltpu.async_copy(h4.at[sidxs[p]], rowss[p], sems[p])

    def wait_fetch(p):
        pltpu.make_async_copy(atab4.at[sidxs[p]], abss[p], sems[p]).wait()
        pltpu.make_async_copy(atab4.at[didxs[p]], abds[p], sems[p]).wait()
        pltpu.make_async_copy(h4.at[sidxs[p]], rowss[p], sems[p]).wait()

    def process(p):
        for j in range(K // 16):
            e16 = j * 16 + iota
            m16 = plsc.load_gather(ebufs[p], [e16, full(2)])
            asv = plsc.load_gather(abss[p], [e16, full(0)])
            adv = plsc.load_gather(abds[p], [e16, full(1)])
            wv = jnp.exp(_leaky(asv + adv))
            wv = jnp.where(m16 <= 1, wv, 0.0)
            plsc.store_scatter(wbufs[p], [e16, full(0)], wv)

        def ebody(eh, carry2):
            for u in range(4):
                e = eh * 4 + u
                fe = jnp.full((16,), e, jnp.int32)
                wspl = plsc.load_gather(wbufs[p], [fe, full(0)])
                rowss[p][e, pl.ds(0, 16)] = rowss[p][e, pl.ds(0, 16)] * wspl
            return carry2
        lax.fori_loop(0, K // 4, ebody, 0)

        pltpu.sync_copy(rowss[p], num_acc.at[didxs[p]], add=True)
        pltpu.sync_copy(wbufs[p], den_acc.at[didxs[p]], add=True)

    def chunk_body(it, carry):
        fetch(it * 2, 0)
        fetch(it * 2 + 1, 1)
        wait_fetch(0)
        process(0)
        wait_fetch(1)
        process(1)
        return carry
    lax.fori_loop(0, npairs, chunk_body, 0)
    plsc.subcore_barrier()

    r0 = sid * RPT
    pltpu.sync_copy(num_acc.at[pl.ds(r0, RPT)], num_out.at[cid, pl.ds(r0, RPT)])
    pltpu.sync_copy(den_acc.at[pl.ds(r0, RPT)], den_out.at[cid, pl.ds(r0, RPT)])


def _edges4_call(h4, atab4, epack, zn4, zd):
    mesh = plsc.VectorSubcoreMesh(core_axis_name="c", subcore_axis_name="s")
    f = pl.kernel(
        _edges4_body,
        mesh=mesh,
        out_type=[
            jax.ShapeDtypeStruct((2, N, 16), jnp.float32),
            jax.ShapeDtypeStruct((2, N, 4), jnp.float32),
        ],
        scratch_types=(
            [pltpu.VMEM((K, 4), jnp.int32)] * 2
            + [pltpu.VMEM((K,), jnp.int32)] * 4
            + [pltpu.VMEM((K, 8), jnp.float32)] * 4
            + [pltpu.VMEM((K, 16), jnp.float32)] * 2
            + [pltpu.VMEM((K, 4), jnp.float32)] * 2
            + [
                pltpu.VMEM_SHARED((N, 16), jnp.float32),
                pltpu.VMEM_SHARED((N, 4), jnp.float32),
                pltpu.SemaphoreType.DMA,
                pltpu.SemaphoreType.DMA,
            ]
        ),
        compiler_params=_SC_PARAMS,
    )
    return f(h4, atab4, epack, zn4, zd)


def _post_body(num0_ref, num1_ref, den0_ref, den1_ref, h4_ref, atab4_ref,
               bl_ref, out_ref):
    at = atab4_ref[...]
    wsl = jnp.exp(_leaky(at[:, 0] + at[:, 1]))
    num = num0_ref[0] + num1_ref[0] + h4_ref[...] * wsl[:, None]
    den = den0_ref[0][:, 0] + den1_ref[0][:, 0] + wsl
    o = num / (den[:, None] + 1e-16) + bl_ref[...]
    m = jnp.max(o, axis=1, keepdims=True)
    p = jnp.exp(o - m)
    out_ref[...] = p / jnp.sum(p, axis=1, keepdims=True)


def _post_call(num4, den4, h4, atab4, b_last):
    nb = 10
    bs = N // nb
    return pl.pallas_call(
        _post_body,
        grid=(nb,),
        in_specs=[
            pl.BlockSpec((1, bs, 16), lambda i: (0, i, 0)),
            pl.BlockSpec((1, bs, 16), lambda i: (1, i, 0)),
            pl.BlockSpec((1, bs, 4), lambda i: (0, i, 0)),
            pl.BlockSpec((1, bs, 4), lambda i: (1, i, 0)),
            pl.BlockSpec((bs, 16), lambda i: (i, 0)),
            pl.BlockSpec((bs, 8), lambda i: (i, 0)),
            pl.BlockSpec((1, 16), lambda i: (0, 0)),
        ],
        out_specs=pl.BlockSpec((bs, 16), lambda i: (i, 0)),
        out_shape=jax.ShapeDtypeStruct((N, 16), jnp.float32),
    )(num4, num4, den4, den4, h4, atab4, b_last)


def kernel(x, edge_index, is_reversed, W_st, a_src_st, a_dst_st, b_st,
           W_ts, a_src_ts, a_dst_ts, b_ts, W_c, a_src_c, a_dst_c, b_c,
           W_last, a_src_last, a_dst_last, b_last):
    W3 = jnp.stack([W_st, W_ts, W_c])
    a_src3 = jnp.stack([a_src_st.reshape(1, D), a_src_ts.reshape(1, D),
                        a_src_c.reshape(1, D)])
    a_dst3 = jnp.stack([a_dst_st.reshape(1, D), a_dst_ts.reshape(1, D),
                        a_dst_c.reshape(1, D)])
    bcat = jnp.concatenate([b_st, b_ts, b_c]).reshape(1, 3 * D)

    src = edge_index[0].astype(jnp.int32)
    dst = edge_index[1].astype(jnp.int32)
    mrev = is_reversed.astype(jnp.int32)
    pad = E_PAD + K - E
    src = jnp.concatenate([src, jnp.zeros((pad,), jnp.int32)])
    dst = jnp.concatenate([dst, jnp.zeros((pad,), jnp.int32)])
    mrev = jnp.concatenate([mrev, jnp.full((pad,), 2, jnp.int32)])
    epack = jnp.stack([src, dst, mrev, jnp.zeros_like(src)], axis=1)

    zn = jnp.zeros((125, D), jnp.float32)
    zd = jnp.zeros((125, 4), jnp.float32)
    zn4 = jnp.zeros((125, 16), jnp.float32)

    hb3, atab3 = _pre_call(x, W3, a_src3, a_dst3)
    num, den = _edges3_call(hb3, atab3, epack, zn, zd)
    h4, atab4 = _mid_call(num.reshape(6, N, D), den.reshape(6, N, 4),
                          hb3, atab3, bcat, W_last,
                          a_src_last.reshape(1, 16), a_dst_last.reshape(1, 16))
    num4, den4 = _edges4_call(h4, atab4, epack, zn4, zd)
    return _post_call(num4, den4, h4, atab4, b_last.reshape(1, 16))


# speedup vs baseline: 1.2337x; 1.2337x over previous
import jax
import jax.numpy as jnp
from jax import lax
from jax.experimental import pallas as pl
from jax.experimental.pallas import tpu as pltpu
from jax.experimental.pallas import tpu_sc as plsc

N = 10000
E = 320000
D = 128
H = 2
C = 64
NUM_CLASSES = 16

K = 128
E_PAD = 327680
EPT = E_PAD // 32
NCHUNK = EPT // K
RPT = N // 16
CH0 = 102
CH1 = 58
NEG_SLOPE = 0.2

_SC_PARAMS = pltpu.CompilerParams(
    use_tc_tiling_on_sc=False, needs_layout_passes=False)


def _leaky(x):
    return jnp.where(x >= 0, x, NEG_SLOPE * x)


def _pre_body(x_ref, w_ref, asv_ref, adv_ref, h_ref, atab_ref):
    hb = jnp.dot(x_ref[...], w_ref[0], preferred_element_type=jnp.float32)
    h_ref[...] = hb
    asv = asv_ref[0]
    adv = adv_ref[0]
    cols = []
    for g in range(2):
        sl = slice(g * 64, (g + 1) * 64)
        cols.append(jnp.sum(hb[:, sl] * asv[:, sl], axis=1))
    for g in range(2):
        sl = slice(g * 64, (g + 1) * 64)
        cols.append(jnp.sum(hb[:, sl] * adv[:, sl], axis=1))
    z = jnp.zeros_like(cols[0])
    atab_ref[...] = jnp.stack(cols + [z, z, z, z], axis=1)


def _pre_call(x, W3, a_src3, a_dst3):
    nb = 10
    bs = N // nb
    return pl.pallas_call(
        _pre_body,
        grid=(3 * nb,),
        in_specs=[
            pl.BlockSpec((bs, D), lambda i: (i % nb, 0)),
            pl.BlockSpec((1, D, D), lambda i: (i // nb, 0, 0)),
            pl.BlockSpec((1, 1, D), lambda i: (i // nb, 0, 0)),
            pl.BlockSpec((1, 1, D), lambda i: (i // nb, 0, 0)),
        ],
        out_specs=[
            pl.BlockSpec((bs, D), lambda i: (i, 0)),
            pl.BlockSpec((bs, 8), lambda i: (i, 0)),
        ],
        out_shape=[
            jax.ShapeDtypeStruct((3 * N, D), jnp.float32),
            jax.ShapeDtypeStruct((3 * N, 8), jnp.float32),
        ],
    )(x, W3, a_src3, a_dst3)


def _edges3_body(hb3, atab3, epack, zn, zd,
                 num_out, den_out,
                 ebuf0, ebuf1, sidx0, sidx1, didx0, didx1, didxa0, didxa1,
                 abs0, abs1, abd0, abd1, rows0, rows1, wbuf0, wbuf1,
                 num_acc, den_acc, sem0, sem1):
    cid = lax.axis_index("c")
    sid = lax.axis_index("s")
    iota = lax.iota(jnp.int32, 16)
    cid32 = cid.astype(jnp.int32)
    base_chunk = sid * (CH0 + CH1) + cid32 * CH0
    npairs = jnp.where(cid32 == 0, CH0 // 2, CH1 // 2)
    full = lambda v: jnp.full((16,), v, jnp.int32)
    ebufs = [ebuf0, ebuf1]
    sidxs = [sidx0, sidx1]
    didxs = [didx0, didx1]
    didxas = [didxa0, didxa1]
    abss = [abs0, abs1]
    abds = [abd0, abd1]
    rowss = [rows0, rows1]
    wbufs = [wbuf0, wbuf1]
    sems = [sem0, sem1]

    for br in range(3):
        def zbody(i, carry):
            r0 = sid * RPT + i * 125
            pltpu.sync_copy(zn, num_acc.at[pl.ds(r0, 125)])
            pltpu.sync_copy(zd, den_acc.at[pl.ds(r0, 125)])
            return carry
        lax.fori_loop(0, RPT // 125, zbody, 0)
        plsc.subcore_barrier()

        def fetch(c, p):
            base = (base_chunk + c) * K
            pltpu.sync_copy(epack.at[pl.ds(base, K)], ebufs[p])
            for j in range(K // 16):
                e16 = j * 16 + iota
                s16 = plsc.load_gather(ebufs[p], [e16, full(0)])
                d16 = plsc.load_gather(ebufs[p], [e16, full(1)])
                plsc.store_scatter(sidxs[p], [e16], s16 + br * N)
                plsc.store_scatter(didxs[p], [e16], d16)
                plsc.store_scatter(didxas[p], [e16], d16 + br * N)
            pltpu.async_copy(atab3.at[sidxs[p]], abss[p], sems[p])
            pltpu.async_copy(atab3.at[didxas[p]], abds[p], sems[p])
            pltpu.async_copy(hb3.at[sidxs[p]], rowss[p], sems[p])

        def wait_fetch(p):
            pltpu.make_async_copy(atab3.at[sidxs[p]], abss[p], sems[p]).wait()
            pltpu.make_async_copy(atab3.at[didxas[p]], abds[p], sems[p]).wait()
            pltpu.make_async_copy(hb3.at[sidxs[p]], rowss[p], sems[p]).wait()

        def process(p):
            for j in range(K // 16):
                e16 = j * 16 + iota
                m16 = plsc.load_gather(ebufs[p], [e16, full(2)])
                if br == 0:
                    emask = m16 == 0
                elif br == 1:
                    emask = m16 == 1
                else:
                    emask = m16 <= 1
                for g in range(2):
                    asv = plsc.load_gather(abss[p], [e16, full(g)])
                    adv = plsc.load_gather(abds[p], [e16, full(2 + g)])
                    wv = jnp.exp(_leaky(asv + adv))
                    wv = jnp.where(emask, wv, 0.0)
                    plsc.store_scatter(wbufs[p], [e16, full(g)], wv)

            def ebody(eh, carry2):
                for u in range(4):
                    e = eh * 4 + u
                    fe = jnp.full((16,), e, jnp.int32)
                    for g in range(2):
                        wspl = plsc.load_gather(wbufs[p], [fe, full(g)])
                        for q in range(4):
                            sl = pl.ds(g * 64 + q * 16, 16)
                            rowss[p][e, sl] = rowss[p][e, sl] * wspl
                return carry2
            lax.fori_loop(0, K // 4, ebody, 0)

            pltpu.sync_copy(rowss[p], num_acc.at[didxs[p]], add=True)
            pltpu.sync_copy(wbufs[p], den_acc.at[didxs[p]], add=True)

        def chunk_body(it, carry):
            fetch(it * 2, 0)
            fetch(it * 2 + 1, 1)
            wait_fetch(0)
            process(0)
            wait_fetch(1)
            process(1)
            return carry
        lax.fori_loop(0, npairs, chunk_body, 0)
        plsc.subcore_barrier()

        r0 = sid * RPT
        pltpu.sync_copy(num_acc.at[pl.ds(r0, RPT)],
                        num_out.at[br, cid, pl.ds(r0, RPT)])
        pltpu.sync_copy(den_acc.at[pl.ds(r0, RPT)],
                        den_out.at[br, cid, pl.ds(r0, RPT)])
        plsc.subcore_barrier()


def _edges3_call(hb3, atab3, epack, zn, zd):
    mesh = plsc.VectorSubcoreMesh(core_axis_name="c", subcore_axis_name="s")
    f = pl.kernel(
        _edges3_body,
        mesh=mesh,
        out_type=[
            jax.ShapeDtypeStruct((3, 2, N, D), jnp.float32),
            jax.ShapeDtypeStruct((3, 2, N, 4), jnp.float32),
        ],
        scratch_types=(
            [pltpu.VMEM((K, 4), jnp.int32)] * 2
            + [pltpu.VMEM((K,), jnp.int32)] * 6
            + [pltpu.VMEM((K, 8), jnp.float32)] * 4
            + [pltpu.VMEM((K, D), jnp.float32)] * 2
            + [pltpu.VMEM((K, 4), jnp.float32)] * 2
            + [
                pltpu.VMEM_SHARED((N, D), jnp.float32),
                pltpu.VMEM_SHARED((N, 4), jnp.float32),
                pltpu.SemaphoreType.DMA,
                pltpu.SemaphoreType.DMA,
            ]
        ),
        compiler_params=_SC_PARAMS,
    )
    return f(hb3, atab3, epack, zn, zd)


def _mid_body(n0_ref, n1_ref, n2_ref, n3_ref, n4_ref, n5_ref,
              d0_ref, d1_ref, d2_ref, d3_ref, d4_ref, d5_ref,
              h0_ref, h1_ref, h2_ref, at0_ref, at1_ref, at2_ref,
              bcat_ref, wl_ref, asl_ref, adl_ref,
              h4_ref, atab4_ref):
    nrefs = [n0_ref, n1_ref, n2_ref, n3_ref, n4_ref, n5_ref]
    drefs = [d0_ref, d1_ref, d2_ref, d3_ref, d4_ref, d5_ref]
    hrefs = [h0_ref, h1_ref, h2_ref]
    atrefs = [at0_ref, at1_ref, at2_ref]
    branches = []
    for br in range(3):
        at = atrefs[br][...]
        wsl = jnp.exp(_leaky(at[:, 0:2] + at[:, 2:4]))
        bs = wsl.shape[0]
        wslx = jnp.broadcast_to(wsl[:, :, None], (bs, 2, 64)).reshape(bs, D)
        numf = nrefs[2 * br][0] + nrefs[2 * br + 1][0] + hrefs[br][...] * wslx
        denf = drefs[2 * br][0][:, 0:2] + drefs[2 * br + 1][0][:, 0:2] + wsl
        denx = jnp.broadcast_to(denf[:, :, None], (bs, 2, 64)).reshape(bs, D)
        outb = numf / (denx + 1e-16) \
            + bcat_ref[0, br * D:(br + 1) * D][None, :]
        branches.append(jnp.where(outb > 0, outb, jnp.exp(outb) - 1.0))
    xc = jnp.concatenate(branches, axis=1)
    h4 = jnp.dot(xc, wl_ref[...], preferred_element_type=jnp.float32)
    h4_ref[...] = h4
    asrc4 = jnp.sum(h4 * asl_ref[...], axis=1)
    adst4 = jnp.sum(h4 * adl_ref[...], axis=1)
    z = jnp.zeros_like(asrc4)
    atab4_ref[...] = jnp.stack([asrc4, adst4, z, z, z, z, z, z], axis=1)


def _mid_call(num6, den6, hb3, atab3, bcat, W_last, a_src_last, a_dst_last):
    nb = 10
    bs = N // nb
    nspec = [pl.BlockSpec((1, bs, D), (lambda k: lambda i: (k, i, 0))(k))
             for k in range(6)]
    dspec = [pl.BlockSpec((1, bs, 4), (lambda k: lambda i: (k, i, 0))(k))
             for k in range(6)]
    hspec = [pl.BlockSpec((bs, D), (lambda b: lambda i: (i + b * nb, 0))(b))
             for b in range(3)]
    aspec = [pl.BlockSpec((bs, 8), (lambda b: lambda i: (i + b * nb, 0))(b))
             for b in range(3)]
    return pl.pallas_call(
        _mid_body,
        grid=(nb,),
        in_specs=nspec + dspec + hspec + aspec + [
            pl.BlockSpec((1, 3 * D), lambda i: (0, 0)),
            pl.BlockSpec((3 * D, 16), lambda i: (0, 0)),
            pl.BlockSpec((1, 16), lambda i: (0, 0)),
            pl.BlockSpec((1, 16), lambda i: (0, 0)),
        ],
        out_specs=[
            pl.BlockSpec((bs, 16), lambda i: (i, 0)),
            pl.BlockSpec((bs, 8), lambda i: (i, 0)),
        ],
        out_shape=[
            jax.ShapeDtypeStruct((N, 16), jnp.float32),
            jax.ShapeDtypeStruct((N, 8), jnp.float32),
        ],
    )(*([num6] * 6), *([den6] * 6), *([hb3] * 3), *([atab3] * 3),
      bcat, W_last, a_src_last, a_dst_last)


def _edges4_body(h4, atab4, epack, zn4, zd,
                 num_out, den_out,
                 ebuf0, ebuf1, sidx0, sidx1, didx0, didx1,
                 abs0, abs1, abd0, abd1, rows0, rows1, wbuf0, wbuf1,
                 num_acc, den_acc, sem0, sem1):
    cid = lax.axis_index("c")
    sid = lax.axis_index("s")
    iota = lax.iota(jnp.int32, 16)
    cid32 = cid.astype(jnp.int32)
    base_chunk = sid * (CH0 + CH1) + cid32 * CH0
    npairs = jnp.where(cid32 == 0, CH0 // 2, CH1 // 2)
    full = lambda v: jnp.full((16,), v, jnp.int32)
    ebufs = [ebuf0, ebuf1]
    sidxs = [sidx0, sidx1]
    didxs = [didx0, didx1]
    abss = [abs0, abs1]
    abds = [abd0, abd1]
    rowss = [rows0, rows1]
    wbufs = [wbuf0, wbuf1]
    sems = [sem0, sem1]

    def zbody(i, carry):
        r0 = sid * RPT + i * 125
        pltpu.sync_copy(zn4, num_acc.at[pl.ds(r0, 125)])
        pltpu.sync_copy(zd, den_acc.at[pl.ds(r0, 125)])
        return carry
    lax.fori_loop(0, RPT // 125, zbody, 0)
    plsc.subcore_barrier()

    def fetch(c, p):
        base = (base_chunk + c) * K
        pltpu.sync_copy(epack.at[pl.ds(base, K)], ebufs[p])
        for j in range(K // 16):
            e16 = j * 16 + iota
            s16 = plsc.load_gather(ebufs[p], [e16, full(0)])
            d16 = plsc.load_gather(ebufs[p], [e16, full(1)])
            plsc.store_scatter(sidxs[p], [e16], s16)
            plsc.store_scatter(didxs[p], [e16], d16)
        pltpu.async_copy(atab4.at[sidxs[p]], abss[p], sems[p])
        pltpu.async_copy(atab4.at[didxs[p]], abds[p], sems[p])
        pltpu.async_copy(h4.at[sidxs[p]], rowss[p], sems[p])

    def wait_fetch(p):
        pltpu.make_async_copy(atab4.at[sidxs[p]], abss[p], sems[p]).wait()
        pltpu.make_async_copy(atab4.at[didxs[p]], abds[p], sems[p]).wait()
        pltpu.make_async_copy(h4.at[sidxs[p]], rowss[p], sems[p]).wait()

    def process(p):
        for j in range(K // 16):
            e16 = j * 16 + iota
            m16 = plsc.load_gather(ebufs[p], [e16, full(2)])
            asv = plsc.load_gather(abss[p], [e16, full(0)])
            adv = plsc.load_gather(abds[p], [e16, full(1)])
            wv = jnp.exp(_leaky(asv + adv))
            wv = jnp.where(m16 <= 1, wv, 0.0)
            plsc.store_scatter(wbufs[p], [e16, full(0)], wv)

        def ebody(eh, carry2):
            for u in range(4):
                e = eh * 4 + u
                fe = jnp.full((16,), e, jnp.int32)
                wspl = plsc.load_gather(wbufs[p], [fe, full(0)])
                rowss[p][e, pl.ds(0, 16)] = rowss[p][e, pl.ds(0, 16)] * wspl
            return carry2
        lax.fori_loop(0, K // 4, ebody, 0)

        pltpu.sync_copy(rowss[p], num_acc.at[didxs[p]], add=True)
        pltpu.sync_copy(wbufs[p], den_acc.at[didxs[p]], add=True)

    def chunk_body(it, carry):
        fetch(it * 2, 0)
        fetch(it * 2 + 1, 1)
        wait_fetch(0)
        process(0)
        wait_fetch(1)
        process(1)
        return carry
    lax.fori_loop(0, npairs, chunk_body, 0)
    plsc.subcore_barrier()

    r0 = sid * RPT
    pltpu.sync_copy(num_acc.at[pl.ds(r0, RPT)], num_out.at[cid, pl.ds(r0, RPT)])
    pltpu.sync_copy(den_acc.at[pl.ds(r0, RPT)], den_out.at[cid, pl.ds(r0, RPT)])


def _edges4_call(h4, atab4, epack, zn4, zd):
    mesh = plsc.VectorSubcoreMesh(core_axis_name="c", subcore_axis_name="s")
    f = pl.kernel(
        _edges4_body,
        mesh=mesh,
        out_type=[
            jax.ShapeDtypeStruct((2, N, 16), jnp.float32),
            jax.ShapeDtypeStruct((2, N, 4), jnp.float32),
        ],
        scratch_types=(
            [pltpu.VMEM((K, 4), jnp.int32)] * 2
            + [pltpu.VMEM((K,), jnp.int32)] * 4
            + [pltpu.VMEM((K, 8), jnp.float32)] * 4
            + [pltpu.VMEM((K, 16), jnp.float32)] * 2
            + [pltpu.VMEM((K, 4), jnp.float32)] * 2
            + [
                pltpu.VMEM_SHARED((N, 16), jnp.float32),
                pltpu.VMEM_SHARED((N, 4), jnp.float32),
                pltpu.SemaphoreType.DMA,
                pltpu.SemaphoreType.DMA,
            ]
        ),
        compiler_params=_SC_PARAMS,
    )
    return f(h4, atab4, epack, zn4, zd)


def _post_body(num0_ref, num1_ref, den0_ref, den1_ref, h4_ref, atab4_ref,
               bl_ref, out_ref):
    at = atab4_ref[...]
    wsl = jnp.exp(_leaky(at[:, 0] + at[:, 1]))
    num = num0_ref[0] + num1_ref[0] + h4_ref[...] * wsl[:, None]
    den = den0_ref[0][:, 0] + den1_ref[0][:, 0] + wsl
    o = num / (den[:, None] + 1e-16) + bl_ref[...]
    m = jnp.max(o, axis=1, keepdims=True)
    p = jnp.exp(o - m)
    out_ref[...] = p / jnp.sum(p, axis=1, keepdims=True)


def _post_call(num4, den4, h4, atab4, b_last):
    nb = 10
    bs = N // nb
    return pl.pallas_call(
        _post_body,
        grid=(nb,),
        in_specs=[
            pl.BlockSpec((1, bs, 16), lambda i: (0, i, 0)),
            pl.BlockSpec((1, bs, 16), lambda i: (1, i, 0)),
            pl.BlockSpec((1, bs, 4), lambda i: (0, i, 0)),
            pl.BlockSpec((1, bs, 4), lambda i: (1, i, 0)),
            pl.BlockSpec((bs, 16), lambda i: (i, 0)),
            pl.BlockSpec((bs, 8), lambda i: (i, 0)),
            pl.BlockSpec((1, 16), lambda i: (0, 0)),
        ],
        out_specs=pl.BlockSpec((bs, 16), lambda i: (i, 0)),
        out_shape=jax.ShapeDtypeStruct((N, 16), jnp.float32),
    )(num4, num4, den4, den4, h4, atab4, b_last)


def kernel(x, edge_index, is_reversed, W_st, a_src_st, a_dst_st, b_st,
           W_ts, a_src_ts, a_dst_ts, b_ts, W_c, a_src_c, a_dst_c, b_c,
           W_last, a_src_last, a_dst_last, b_last):
    W3 = jnp.stack([W_st, W_ts, W_c])
    a_src3 = jnp.stack([a_src_st.reshape(1, D), a_src_ts.reshape(1, D),
                        a_src_c.reshape(1, D)])
    a_dst3 = jnp.stack([a_dst_st.reshape(1, D), a_dst_ts.reshape(1, D),
                        a_dst_c.reshape(1, D)])
    bcat = jnp.concatenate([b_st, b_ts, b_c]).reshape(1, 3 * D)

    src = edge_index[0].astype(jnp.int32)
    dst = edge_index[1].astype(jnp.int32)
    mrev = is_reversed.astype(jnp.int32)
    pad = E_PAD + K - E
    src = jnp.concatenate([src, jnp.zeros((pad,), jnp.int32)])
    dst = jnp.concatenate([dst, jnp.zeros((pad,), jnp.int32)])
    mrev = jnp.concatenate([mrev, jnp.full((pad,), 2, jnp.int32)])
    epack = jnp.stack([src, dst, mrev, jnp.zeros_like(src)], axis=1)

    zn = jnp.zeros((125, D), jnp.float32)
    zd = jnp.zeros((125, 4), jnp.float32)
    zn4 = jnp.zeros((125, 16), jnp.float32)

    hb3, atab3 = _pre_call(x, W3, a_src3, a_dst3)
    num, den = _edges3_call(hb3, atab3, epack, zn, zd)
    h4, atab4 = _mid_call(num.reshape(6, N, D), den.reshape(6, N, 4),
                          hb3, atab3, bcat, W_last,
                          a_src_last.reshape(1, 16), a_dst_last.reshape(1, 16))
    num4, den4 = _edges4_call(h4, atab4, epack, zn4, zd)
    return _post_call(num4, den4, h4, atab4, b_last.reshape(1, 16))
